# baseline (speedup 1.0000x reference)
import functools

import jax
import jax.numpy as jnp
from jax import lax
from jax.experimental import pallas as pl
from jax.experimental.pallas import tpu as pltpu
from jax.experimental.pallas import tpu_sc as plsc

_L = 16


def _tree_sum(terms):
    while len(terms) > 1:
        nxt = [terms[i] + terms[i + 1] for i in range(0, len(terms) - 1, 2)]
        if len(terms) % 2:
            nxt.append(terms[-1])
        terms = nxt
    return terms[0]


def _meta_body(x_ref, w_ref, b_ref, o_ref):
    prod = jnp.dot(x_ref[...], w_ref[...], preferred_element_type=jnp.float32)
    o_ref[...] = prod.reshape(o_ref.shape) + b_ref[0]


def _tile_body(nc, rpw, chunk, d,
               user_hbm, item_hbm, mo_hbm, utab_hbm, itab_hbm, w_hbm, out_hbm,
               idx_u, idx_i, u0, i0, u1, i1,
               w_v, mo_v, trans, out_v, stg_sem, w_sem, sem0, sem1):
    wid = lax.axis_index("s") * nc + lax.axis_index("c")
    base = pl.multiple_of(wid * rpw, rpw)

    idx_cps = (
        pltpu.async_copy(user_hbm.at[pl.ds(base, rpw)], idx_u, stg_sem),
        pltpu.async_copy(item_hbm.at[pl.ds(base, rpw)], idx_i, stg_sem),
    )
    w_cps = (
        pltpu.async_copy(w_hbm, w_v, w_sem),
        pltpu.async_copy(mo_hbm.at[pl.ds(base, rpw)], mo_v, w_sem),
    )
    for cp in idx_cps:
        cp.wait()

    lane16 = lax.iota(jnp.int32, _L) * _L
    nchunk = rpw // chunk
    ngroup = chunk // _L
    bufs = [(u0, i0), (u1, i1)]
    sems = [sem0, sem1]

    def start(c, parity):
        cb = pl.multiple_of(c * chunk, chunk)
        ub, ib = bufs[parity]
        sem = sems[parity]
        pltpu.async_copy(utab_hbm.at[idx_u.at[pl.ds(cb, chunk)]], ub, sem)
        pltpu.async_copy(itab_hbm.at[idx_i.at[pl.ds(cb, chunk)]], ib, sem)

    def drain(parity):
        ub, ib = bufs[parity]
        sem = sems[parity]
        pltpu.make_async_copy(utab_hbm.at[idx_u.at[pl.ds(0, chunk)]], ub, sem).wait()
        pltpu.make_async_copy(itab_hbm.at[idx_i.at[pl.ds(0, chunk)]], ib, sem).wait()

    start(0, 0)
    start(1, 1)
    for cp in w_cps:
        cp.wait()
    wv = [w_v[pl.ds(k * _L, _L)] for k in range(d // _L)]

    def compute_chunk(c, parity):
        ub, ib = bufs[parity]
        cb = pl.multiple_of(c * chunk, chunk)

        def group(g, carry, ub=ub, ib=ib, cb=cb):
            gb = g * _L
            half = _L // 2
            for h in range(2):
                hb = gb + h * half
                accs = [None] * half
                for k in range(d // _L):
                    for l in range(half):
                        t = ub[hb + l, pl.ds(k * _L, _L)] * ib[hb + l, pl.ds(k * _L, _L)] * wv[k]
                        accs[l] = t if k == 0 else accs[l] + t
                for l in range(half):
                    trans[pl.ds((h * half + l) * _L, _L)] = accs[l]
            cols = [plsc.load_gather(trans, [lane16 + j]) for j in range(_L)]
            out_v[pl.ds(cb + gb, _L)] = _tree_sum(cols) + mo_v[pl.ds(cb + gb, _L)]
            return carry

        lax.fori_loop(0, ngroup, group, 0)

    def pair_body(p, carry):
        for parity in range(2):
            c = p * 2 + parity
            drain(parity)
            compute_chunk(c, parity)

            @pl.when(c + 2 < nchunk)
            def _(c=c, parity=parity):
                start(c + 2, parity)
        return carry

    lax.fori_loop(0, nchunk // 2, pair_body, 0)

    pltpu.sync_copy(out_v, out_hbm.at[pl.ds(base, rpw)])


def kernel(user, item, item_metadata, user_table, item_table,
           comb_w, comb_b, meta_w, meta_b, global_bias):
    b = user.shape[0]
    d = user_table.shape[1]
    m = item_metadata.shape[1]
    info = plsc.get_sparse_core_info()
    nc, ns = info.num_cores, info.num_subcores
    nw = nc * ns
    rpw = b // nw
    chunk = 128

    blk = 2048
    bias = (comb_b + meta_b + global_bias).astype(jnp.float32)
    meta_out = pl.pallas_call(
        _meta_body,
        grid=(b // blk,),
        in_specs=[
            pl.BlockSpec((blk, m), lambda i: (i, 0)),
            pl.BlockSpec((m, 1), lambda i: (0, 0)),
            pl.BlockSpec(memory_space=pltpu.SMEM),
        ],
        out_specs=pl.BlockSpec((1, 1, blk), lambda i: (i, 0, 0)),
        out_shape=jax.ShapeDtypeStruct((b // blk, 1, blk), jnp.float32),
    )(item_metadata, meta_w, bias).reshape(b)

    w = comb_w.reshape(d)
    mesh = plsc.VectorSubcoreMesh(core_axis_name="c", subcore_axis_name="s")
    kfn = pl.kernel(
        functools.partial(_tile_body, nc, rpw, chunk, d),
        mesh=mesh,
        compiler_params=pltpu.CompilerParams(needs_layout_passes=False),
        out_type=jax.ShapeDtypeStruct((b,), jnp.float32),
        scratch_types=[
            pltpu.VMEM((rpw,), jnp.int32),
            pltpu.VMEM((rpw,), jnp.int32),
            pltpu.VMEM((chunk, d), jnp.float32),
            pltpu.VMEM((chunk, d), jnp.float32),
            pltpu.VMEM((chunk, d), jnp.float32),
            pltpu.VMEM((chunk, d), jnp.float32),
            pltpu.VMEM((d,), jnp.float32),
            pltpu.VMEM((rpw,), jnp.float32),
            pltpu.VMEM((_L * _L,), jnp.float32),
            pltpu.VMEM((rpw,), jnp.float32),
            pltpu.SemaphoreType.DMA,
            pltpu.SemaphoreType.DMA,
            pltpu.SemaphoreType.DMA,
            pltpu.SemaphoreType.DMA,
        ],
    )
    return kfn(user, item, meta_out, user_table, item_table, w)


# speedup vs baseline: 1.3016x; 1.3016x over previous
import functools

import jax
import jax.numpy as jnp
from jax import lax
from jax.experimental import pallas as pl
from jax.experimental.pallas import tpu as pltpu
from jax.experimental.pallas import tpu_sc as plsc

_L = 16


def _tree_sum(terms):
    while len(terms) > 1:
        nxt = [terms[i] + terms[i + 1] for i in range(0, len(terms) - 1, 2)]
        if len(terms) % 2:
            nxt.append(terms[-1])
        terms = nxt
    return terms[0]


def _meta_body(x_ref, w_ref, b_ref, o_ref):
    prod = jnp.dot(x_ref[...], w_ref[...], preferred_element_type=jnp.float32)
    o_ref[...] = prod.reshape(o_ref.shape) + b_ref[0]


def _add_body(a_ref, b_ref, o_ref):
    o_ref[...] = a_ref[...] + b_ref[...]


def _tile_body(nc, rpw, chunk, d,
               user_hbm, item_hbm, utab_hbm, itab_hbm, w_hbm, out_hbm,
               idx_u, idx_i, u0, i0, u1, i1,
               w_v, trans, out_v, stg_sem, w_sem, sem0, sem1):
    wid = lax.axis_index("s") * nc + lax.axis_index("c")
    base = pl.multiple_of(wid * rpw, rpw)

    idx_cps = (
        pltpu.async_copy(user_hbm.at[pl.ds(base, rpw)], idx_u, stg_sem),
        pltpu.async_copy(item_hbm.at[pl.ds(base, rpw)], idx_i, stg_sem),
    )
    w_cps = (
        pltpu.async_copy(w_hbm, w_v, w_sem),
    )
    for cp in idx_cps:
        cp.wait()

    lane16 = lax.iota(jnp.int32, _L) * _L
    nchunk = rpw // chunk
    ngroup = chunk // _L
    bufs = [(u0, i0), (u1, i1)]
    sems = [sem0, sem1]

    def start(c, parity):
        cb = pl.multiple_of(c * chunk, chunk)
        ub, ib = bufs[parity]
        sem = sems[parity]
        pltpu.async_copy(utab_hbm.at[idx_u.at[pl.ds(cb, chunk)]], ub, sem)
        pltpu.async_copy(itab_hbm.at[idx_i.at[pl.ds(cb, chunk)]], ib, sem)

    def drain(parity):
        ub, ib = bufs[parity]
        sem = sems[parity]
        pltpu.make_async_copy(utab_hbm.at[idx_u.at[pl.ds(0, chunk)]], ub, sem).wait()
        pltpu.make_async_copy(itab_hbm.at[idx_i.at[pl.ds(0, chunk)]], ib, sem).wait()

    start(0, 0)
    start(1, 1)
    for cp in w_cps:
        cp.wait()
    wv = [w_v[pl.ds(k * _L, _L)] for k in range(d // _L)]

    def compute_chunk(c, parity):
        ub, ib = bufs[parity]
        cb = pl.multiple_of(c * chunk, chunk)

        def group(g, carry, ub=ub, ib=ib, cb=cb):
            gb = g * _L
            half = _L // 2
            for h in range(2):
                hb = gb + h * half
                accs = [None] * half
                for k in range(d // _L):
                    for l in range(half):
                        t = ub[hb + l, pl.ds(k * _L, _L)] * ib[hb + l, pl.ds(k * _L, _L)] * wv[k]
                        accs[l] = t if k == 0 else accs[l] + t
                for l in range(half):
                    trans[pl.ds((h * half + l) * _L, _L)] = accs[l]
            cols = [plsc.load_gather(trans, [lane16 + j]) for j in range(_L)]
            out_v[pl.ds(cb + gb, _L)] = _tree_sum(cols)
            return carry

        lax.fori_loop(0, ngroup, group, 0)

    def pair_body(p, carry):
        for parity in range(2):
            c = p * 2 + parity
            drain(parity)
            compute_chunk(c, parity)

            @pl.when(c + 2 < nchunk)
            def _(c=c, parity=parity):
                start(c + 2, parity)
        return carry

    lax.fori_loop(0, nchunk // 2, pair_body, 0)

    pltpu.sync_copy(out_v, out_hbm.at[pl.ds(base, rpw)])


def kernel(user, item, item_metadata, user_table, item_table,
           comb_w, comb_b, meta_w, meta_b, global_bias):
    b = user.shape[0]
    d = user_table.shape[1]
    m = item_metadata.shape[1]
    info = plsc.get_sparse_core_info()
    nc, ns = info.num_cores, info.num_subcores
    nw = nc * ns
    rpw = b // nw
    chunk = 128

    bias = (comb_b + meta_b + global_bias).astype(jnp.float32)
    meta_out = pl.pallas_call(
        _meta_body,
        in_specs=[
            pl.BlockSpec((b, m), lambda: (0, 0)),
            pl.BlockSpec((m, 1), lambda: (0, 0)),
            pl.BlockSpec(memory_space=pltpu.SMEM),
        ],
        out_specs=pl.BlockSpec((1, b), lambda: (0, 0)),
        out_shape=jax.ShapeDtypeStruct((1, b), jnp.float32),
    )(item_metadata, meta_w, bias).reshape(b)

    w = comb_w.reshape(d)
    mesh = plsc.VectorSubcoreMesh(core_axis_name="c", subcore_axis_name="s")
    kfn = pl.kernel(
        functools.partial(_tile_body, nc, rpw, chunk, d),
        mesh=mesh,
        compiler_params=pltpu.CompilerParams(needs_layout_passes=False),
        out_type=jax.ShapeDtypeStruct((b,), jnp.float32),
        scratch_types=[
            pltpu.VMEM((rpw,), jnp.int32),
            pltpu.VMEM((rpw,), jnp.int32),
            pltpu.VMEM((chunk, d), jnp.float32),
            pltpu.VMEM((chunk, d), jnp.float32),
            pltpu.VMEM((chunk, d), jnp.float32),
            pltpu.VMEM((chunk, d), jnp.float32),
            pltpu.VMEM((d,), jnp.float32),
            pltpu.VMEM((_L * _L,), jnp.float32),
            pltpu.VMEM((rpw,), jnp.float32),
            pltpu.SemaphoreType.DMA,
            pltpu.SemaphoreType.DMA,
            pltpu.SemaphoreType.DMA,
            pltpu.SemaphoreType.DMA,
        ],
    )
    sc_out = kfn(user, item, user_table, item_table, w)

    return pl.pallas_call(
        _add_body,
        in_specs=[
            pl.BlockSpec((b,), lambda: (0,)),
            pl.BlockSpec((b,), lambda: (0,)),
        ],
        out_specs=pl.BlockSpec((b,), lambda: (0,)),
        out_shape=jax.ShapeDtypeStruct((b,), jnp.float32),
    )(sc_out, meta_out)


# speedup vs baseline: 1.3018x; 1.0001x over previous
import functools

import jax
import jax.numpy as jnp
from jax import lax
from jax.experimental import pallas as pl
from jax.experimental.pallas import tpu as pltpu
from jax.experimental.pallas import tpu_sc as plsc

_L = 16


def _tree_sum(terms):
    while len(terms) > 1:
        nxt = [terms[i] + terms[i + 1] for i in range(0, len(terms) - 1, 2)]
        if len(terms) % 2:
            nxt.append(terms[-1])
        terms = nxt
    return terms[0]


def _meta_body(x_ref, w_ref, b_ref, o_ref):
    prod = jnp.sum(x_ref[...] * w_ref[...], axis=1)
    o_ref[...] = prod.reshape(o_ref.shape) + b_ref[0]


def _add_body(a_ref, b_ref, o_ref):
    o_ref[...] = a_ref[...] + b_ref[...]


def _tile_body(nc, rpw, chunk, d,
               user_hbm, item_hbm, utab_hbm, itab_hbm, w_hbm, out_hbm,
               idx_u, idx_i, u0, i0, u1, i1,
               w_v, trans, out_v, stg_sem, w_sem, sem0, sem1):
    wid = lax.axis_index("s") * nc + lax.axis_index("c")
    base = pl.multiple_of(wid * rpw, rpw)

    idx_cps = (
        pltpu.async_copy(user_hbm.at[pl.ds(base, rpw)], idx_u, stg_sem),
        pltpu.async_copy(item_hbm.at[pl.ds(base, rpw)], idx_i, stg_sem),
    )
    w_cps = (
        pltpu.async_copy(w_hbm, w_v, w_sem),
    )
    for cp in idx_cps:
        cp.wait()

    lane16 = lax.iota(jnp.int32, _L) * _L
    nchunk = rpw // chunk
    ngroup = chunk // _L
    bufs = [(u0, i0), (u1, i1)]
    sems = [sem0, sem1]

    def start(c, parity):
        cb = pl.multiple_of(c * chunk, chunk)
        ub, ib = bufs[parity]
        sem = sems[parity]
        pltpu.async_copy(utab_hbm.at[idx_u.at[pl.ds(cb, chunk)]], ub, sem)
        pltpu.async_copy(itab_hbm.at[idx_i.at[pl.ds(cb, chunk)]], ib, sem)

    def drain(parity):
        ub, ib = bufs[parity]
        sem = sems[parity]
        pltpu.make_async_copy(utab_hbm.at[idx_u.at[pl.ds(0, chunk)]], ub, sem).wait()
        pltpu.make_async_copy(itab_hbm.at[idx_i.at[pl.ds(0, chunk)]], ib, sem).wait()

    start(0, 0)
    start(1, 1)
    for cp in w_cps:
        cp.wait()
    wv = [w_v[pl.ds(k * _L, _L)] for k in range(d // _L)]

    def compute_chunk(c, parity):
        ub, ib = bufs[parity]
        cb = pl.multiple_of(c * chunk, chunk)

        def group(g, carry, ub=ub, ib=ib, cb=cb):
            gb = g * _L
            half = _L // 2
            for h in range(2):
                hb = gb + h * half
                accs = [None] * half
                for k in range(d // _L):
                    for l in range(half):
                        t = ub[hb + l, pl.ds(k * _L, _L)] * ib[hb + l, pl.ds(k * _L, _L)] * wv[k]
                        accs[l] = t if k == 0 else accs[l] + t
                for l in range(half):
                    trans[pl.ds((h * half + l) * _L, _L)] = accs[l]
            cols = [plsc.load_gather(trans, [lane16 + j]) for j in range(_L)]
            out_v[pl.ds(cb + gb, _L)] = _tree_sum(cols)
            return carry

        lax.fori_loop(0, ngroup, group, 0)

    def pair_body(p, carry):
        for parity in range(2):
            c = p * 2 + parity
            drain(parity)
            compute_chunk(c, parity)

            @pl.when(c + 2 < nchunk)
            def _(c=c, parity=parity):
                start(c + 2, parity)
        return carry

    lax.fori_loop(0, nchunk // 2, pair_body, 0)

    pltpu.sync_copy(out_v, out_hbm.at[pl.ds(base, rpw)])


def kernel(user, item, item_metadata, user_table, item_table,
           comb_w, comb_b, meta_w, meta_b, global_bias):
    b = user.shape[0]
    d = user_table.shape[1]
    m = item_metadata.shape[1]
    info = plsc.get_sparse_core_info()
    nc, ns = info.num_cores, info.num_subcores
    nw = nc * ns
    rpw = b // nw
    chunk = 128

    bias = (comb_b + meta_b + global_bias).astype(jnp.float32)
    meta_out = pl.pallas_call(
        _meta_body,
        in_specs=[
            pl.BlockSpec((b, m), lambda: (0, 0)),
            pl.BlockSpec((1, m), lambda: (0, 0)),
            pl.BlockSpec(memory_space=pltpu.SMEM),
        ],
        out_specs=pl.BlockSpec((1, b), lambda: (0, 0)),
        out_shape=jax.ShapeDtypeStruct((1, b), jnp.float32),
    )(item_metadata, meta_w.reshape(1, m), bias).reshape(b)

    w = comb_w.reshape(d)
    mesh = plsc.VectorSubcoreMesh(core_axis_name="c", subcore_axis_name="s")
    kfn = pl.kernel(
        functools.partial(_tile_body, nc, rpw, chunk, d),
        mesh=mesh,
        compiler_params=pltpu.CompilerParams(needs_layout_passes=False),
        out_type=jax.ShapeDtypeStruct((b,), jnp.float32),
        scratch_types=[
            pltpu.VMEM((rpw,), jnp.int32),
            pltpu.VMEM((rpw,), jnp.int32),
            pltpu.VMEM((chunk, d), jnp.float32),
            pltpu.VMEM((chunk, d), jnp.float32),
            pltpu.VMEM((chunk, d), jnp.float32),
            pltpu.VMEM((chunk, d), jnp.float32),
            pltpu.VMEM((d,), jnp.float32),
            pltpu.VMEM((_L * _L,), jnp.float32),
            pltpu.VMEM((rpw,), jnp.float32),
            pltpu.SemaphoreType.DMA,
            pltpu.SemaphoreType.DMA,
            pltpu.SemaphoreType.DMA,
            pltpu.SemaphoreType.DMA,
        ],
    )
    sc_out = kfn(user, item, user_table, item_table, w)

    return pl.pallas_call(
        _add_body,
        in_specs=[
            pl.BlockSpec((b,), lambda: (0,)),
            pl.BlockSpec((b,), lambda: (0,)),
        ],
        out_specs=pl.BlockSpec((b,), lambda: (0,)),
        out_shape=jax.ShapeDtypeStruct((b,), jnp.float32),
    )(sc_out, meta_out)
